# baseline (speedup 1.0000x reference)
import functools

import jax
import jax.numpy as jnp
from jax import lax
from jax.experimental import pallas as pl
from jax.experimental.pallas import tpu as pltpu
from jax.experimental.pallas import tpu_sc as plsc

_NC = 2
_NS = 16
_EMB = 32
_L = 200
_B = 4096
_V = 1000000
_KC = 12
_LCH = 62464
_TAIL = 1024
_VP = _NS * _LCH + _TAIL


@functools.lru_cache(maxsize=None)
def _build():
    mesh = plsc.VectorSubcoreMesh(core_axis_name="c", subcore_axis_name="s",
                                  num_cores=_NC, num_subcores=_NS)
    dpc = _EMB // _NC

    @functools.partial(
        pl.kernel,
        mesh=mesh,
        out_type=(
            jax.ShapeDtypeStruct((_L * _EMB, _B), jnp.float32),
            jax.ShapeDtypeStruct((8, _B), jnp.float32),
        ),
        scratch_types=[
            pltpu.VMEM_SHARED((_VP,), jnp.float32),
            pltpu.VMEM((_KC * _B,), jnp.int32),
            pltpu.VMEM((3 * _B,), jnp.float32),
            pltpu.VMEM((_B,), jnp.int32),
            pltpu.SemaphoreType.DMA,
            pltpu.SemaphoreType.DMA,
        ],
    )
    def gather_kernel(idx_hbm, tab_hbm, tail_hbm, out_hbm, dump_hbm,
                      shr, idxc, gbuf, idx12, gsem, osem):
        cid = lax.axis_index("c")
        sid = lax.axis_index("s")
        d0 = cid * dpc

        def irow(k):
            return idxc.at[pl.ds(pl.multiple_of(k * _B, _B), _B)]

        def slot(j):
            return gbuf.at[pl.ds(pl.multiple_of(j * _B, _B), _B)]

        for k in range(_KC):
            pltpu.sync_copy(idx_hbm.at[sid + _NS * k], irow(k))

        pltpu.async_copy(slot(0), dump_hbm.at[0], osem)
        pltpu.async_copy(slot(1), dump_hbm.at[1], osem)

        def drain_osem():
            pltpu.make_async_copy(slot(0), dump_hbm.at[0], osem).wait()

        def wait_gsem():
            pltpu.make_async_copy(tab_hbm.at[0, pl.ds(0, _B)], slot(0),
                                  gsem).wait()

        for dd in range(dpc):
            d = d0 + dd

            plsc.subcore_barrier()

            off = pl.multiple_of(sid * _LCH, _LCH)
            pltpu.sync_copy(tab_hbm.at[d, pl.ds(off, _LCH)],
                            shr.at[pl.ds(off, _LCH)])

            @pl.when(sid == 0)
            def _(d=d):
                pltpu.sync_copy(tail_hbm.at[d, pl.ds(0, _TAIL)],
                                shr.at[pl.ds(_NS * _LCH, _TAIL)])
            plsc.subcore_barrier()

            drain_osem()
            pltpu.async_copy(shr.at[irow(0)], slot(0), gsem)

            def body(k, carry, d=d):
                drain_osem()
                pltpu.async_copy(shr.at[irow(k)], slot(k % 3), gsem)
                wait_gsem()
                row = (sid + _NS * (k - 1)) * _EMB + d
                pltpu.async_copy(slot((k + 2) % 3), out_hbm.at[row], osem)
                return carry

            lax.fori_loop(1, _KC, body, 0)

            wait_gsem()
            row = (sid + _NS * (_KC - 1)) * _EMB + d
            pltpu.async_copy(slot((_KC - 1) % 3), out_hbm.at[row], osem)

            l_last = sid + _NS * _KC

            @pl.when(l_last < _L)
            def _(d=d, l_last=l_last):
                pltpu.sync_copy(idx_hbm.at[l_last], idx12)
                pltpu.async_copy(shr.at[idx12], slot(0), gsem).wait()
                pltpu.sync_copy(slot(0), out_hbm.at[l_last * _EMB + d])

        drain_osem()
        drain_osem()

    return gather_kernel


def kernel(input, table):
    idxP = input.T
    tP = table.T
    tailT = jnp.pad(tP[:, _NS * _LCH:], ((0, 0), (0, _VP - _V)))
    out2d, _ = _build()(idxP, tP, tailT)
    out3 = out2d.reshape(_L, _EMB, _B)
    return jnp.transpose(out3, (2, 0, 1))


# speedup vs baseline: 1.0487x; 1.0487x over previous
import functools

import jax
import jax.numpy as jnp
from jax import lax
from jax.experimental import pallas as pl
from jax.experimental.pallas import tpu as pltpu
from jax.experimental.pallas import tpu_sc as plsc

_NC = 2
_NS = 16
_EMB = 32
_L = 200
_B = 4096
_V = 1000000
_KC = 12
_HB = _B // 2
_LCH = 62464
_TAIL = 1024
_VP = _NS * _LCH + _TAIL


@functools.lru_cache(maxsize=None)
def _build():
    mesh = plsc.VectorSubcoreMesh(core_axis_name="c", subcore_axis_name="s",
                                  num_cores=_NC, num_subcores=_NS)
    dpc = _EMB // _NC

    @functools.partial(
        pl.kernel,
        mesh=mesh,
        out_type=(
            jax.ShapeDtypeStruct((_L * _EMB, _B), jnp.float32),
            jax.ShapeDtypeStruct((8, _B), jnp.float32),
        ),
        scratch_types=[
            pltpu.VMEM_SHARED((_VP,), jnp.float32),
            pltpu.VMEM((_KC * _B,), jnp.int32),
            pltpu.VMEM((3 * _B,), jnp.float32),
            pltpu.VMEM((_HB,), jnp.int32),
            pltpu.VMEM((_HB,), jnp.float32),
            pltpu.SemaphoreType.DMA,
            pltpu.SemaphoreType.DMA,
            pltpu.SemaphoreType.DMA,
        ],
    )
    def gather_kernel(idx_hbm, tab_hbm, tail_hbm, out_hbm, dump_hbm,
                      shr, idxc, gbuf, ihalf, ghalf, gsem, osem, osem2):
        cid = lax.axis_index("c")
        sid = lax.axis_index("s")
        d0 = cid * dpc

        def irow(k):
            return idxc.at[pl.ds(pl.multiple_of(k * _B, _B), _B)]

        def slot(j):
            return gbuf.at[pl.ds(pl.multiple_of(j * _B, _B), _B)]

        for k in range(_KC):
            pltpu.sync_copy(idx_hbm.at[sid + _NS * k], irow(k))
        lh = _NS * _KC + sid // 2
        bho = pl.multiple_of((sid % 2) * _HB, _HB)
        pltpu.sync_copy(idx_hbm.at[lh, pl.ds(bho, _HB)], ihalf)

        pltpu.async_copy(slot(0), dump_hbm.at[0], osem)
        pltpu.async_copy(slot(1), dump_hbm.at[1], osem)
        pltpu.async_copy(ghalf, dump_hbm.at[2, pl.ds(0, _HB)], osem2)

        def drain_osem():
            pltpu.make_async_copy(slot(0), dump_hbm.at[0], osem).wait()

        def wait_gsem():
            pltpu.make_async_copy(tab_hbm.at[0, pl.ds(0, _B)], slot(0),
                                  gsem).wait()

        for dd in range(dpc):
            d = d0 + dd

            plsc.subcore_barrier()

            off = pl.multiple_of(sid * _LCH, _LCH)
            pltpu.sync_copy(tab_hbm.at[d, pl.ds(off, _LCH)],
                            shr.at[pl.ds(off, _LCH)])

            @pl.when(sid == 0)
            def _(d=d):
                pltpu.sync_copy(tail_hbm.at[d, pl.ds(0, _TAIL)],
                                shr.at[pl.ds(_NS * _LCH, _TAIL)])
            plsc.subcore_barrier()

            drain_osem()
            pltpu.async_copy(shr.at[irow(0)], slot(0), gsem)

            def body(k, carry, d=d):
                drain_osem()
                pltpu.async_copy(shr.at[irow(k)], slot(k % 3), gsem)
                wait_gsem()
                row = (sid + _NS * (k - 1)) * _EMB + d
                pltpu.async_copy(slot((k + 2) % 3), out_hbm.at[row], osem)
                return carry

            lax.fori_loop(1, _KC, body, 0)

            wait_gsem()
            row = (sid + _NS * (_KC - 1)) * _EMB + d
            pltpu.async_copy(slot((_KC - 1) % 3), out_hbm.at[row], osem)

            pltpu.make_async_copy(ghalf, dump_hbm.at[2, pl.ds(0, _HB)],
                                  osem2).wait()
            pltpu.async_copy(shr.at[ihalf], ghalf, gsem)
            pltpu.make_async_copy(tab_hbm.at[0, pl.ds(0, _HB)], ghalf,
                                  gsem).wait()
            pltpu.async_copy(ghalf,
                             out_hbm.at[lh * _EMB + d, pl.ds(bho, _HB)],
                             osem2)

        drain_osem()
        drain_osem()
        pltpu.make_async_copy(ghalf, dump_hbm.at[2, pl.ds(0, _HB)],
                              osem2).wait()

    return gather_kernel


def kernel(input, table):
    idxP = input.T
    tP = table.T
    tailT = jnp.pad(tP[:, _NS * _LCH:], ((0, 0), (0, _VP - _V)))
    out2d, _ = _build()(idxP, tP, tailT)
    out3 = out2d.reshape(_L, _EMB, _B)
    return jnp.transpose(out3, (2, 0, 1))


# speedup vs baseline: 1.0616x; 1.0123x over previous
import functools

import jax
import jax.numpy as jnp
from jax import lax
from jax.experimental import pallas as pl
from jax.experimental.pallas import tpu as pltpu
from jax.experimental.pallas import tpu_sc as plsc

_NC = 2
_NS = 16
_EMB = 32
_L = 200
_B = 4096
_V = 1000000
_KC = 12
_HB = _B // 2
_LCH = 62464
_TAIL = 1024
_VP = _NS * _LCH + _TAIL


@functools.lru_cache(maxsize=None)
def _build():
    mesh = plsc.VectorSubcoreMesh(core_axis_name="c", subcore_axis_name="s",
                                  num_cores=_NC, num_subcores=_NS)
    dpc = _EMB // _NC

    @functools.partial(
        pl.kernel,
        mesh=mesh,
        out_type=(
            jax.ShapeDtypeStruct((_L * _EMB, _B), jnp.float32),
            jax.ShapeDtypeStruct((8, _B), jnp.float32),
        ),
        scratch_types=[
            pltpu.VMEM_SHARED((_VP,), jnp.float32),
            pltpu.VMEM((_KC * _B,), jnp.int32),
            pltpu.VMEM((3 * _B,), jnp.float32),
            pltpu.VMEM((_HB,), jnp.int32),
            pltpu.VMEM((_HB,), jnp.float32),
            pltpu.SemaphoreType.DMA,
            pltpu.SemaphoreType.DMA,
            pltpu.SemaphoreType.DMA,
            pltpu.SemaphoreType.DMA,
        ],
    )
    def gather_kernel(idx_hbm, tab_hbm, tail_hbm, out_hbm, dump_hbm,
                      shr, idxc, gbuf, ihalf, ghalf, gsem, gsemb, osem, osem2):
        cid = lax.axis_index("c")
        sid = lax.axis_index("s")
        d0 = cid * dpc

        def irow(k):
            return idxc.at[pl.ds(pl.multiple_of(k * _B, _B), _B)]

        def slot(j):
            return gbuf.at[pl.ds(pl.multiple_of(j * _B, _B), _B)]

        for k in range(_KC):
            pltpu.sync_copy(idx_hbm.at[sid + _NS * k], irow(k))
        lh = _NS * _KC + sid // 2
        bho = pl.multiple_of((sid % 2) * _HB, _HB)
        pltpu.sync_copy(idx_hbm.at[lh, pl.ds(bho, _HB)], ihalf)

        pltpu.async_copy(slot(0), dump_hbm.at[0], osem)
        pltpu.async_copy(slot(1), dump_hbm.at[1], osem)
        pltpu.async_copy(ghalf, dump_hbm.at[2, pl.ds(0, _HB)], osem2)

        def drain_osem():
            pltpu.make_async_copy(slot(0), dump_hbm.at[0], osem).wait()

        def fire_gather(k, j):
            pltpu.async_copy(shr.at[irow(k).at[pl.ds(0, _HB)]],
                             slot(j).at[pl.ds(0, _HB)], gsem)
            pltpu.async_copy(shr.at[irow(k).at[pl.ds(_HB, _HB)]],
                             slot(j).at[pl.ds(_HB, _HB)], gsemb)

        def wait_gsem():
            pltpu.make_async_copy(tab_hbm.at[0, pl.ds(0, _HB)], ghalf,
                                  gsem).wait()
            pltpu.make_async_copy(tab_hbm.at[0, pl.ds(0, _HB)], ghalf,
                                  gsemb).wait()

        for dd in range(dpc):
            d = d0 + dd

            plsc.subcore_barrier()

            off = pl.multiple_of(sid * _LCH, _LCH)
            pltpu.sync_copy(tab_hbm.at[d, pl.ds(off, _LCH)],
                            shr.at[pl.ds(off, _LCH)])

            @pl.when(sid == 0)
            def _(d=d):
                pltpu.sync_copy(tail_hbm.at[d, pl.ds(0, _TAIL)],
                                shr.at[pl.ds(_NS * _LCH, _TAIL)])
            plsc.subcore_barrier()

            drain_osem()
            fire_gather(0, 0)

            def body(k, carry, d=d):
                drain_osem()
                fire_gather(k, k % 3)
                wait_gsem()
                row = (sid + _NS * (k - 1)) * _EMB + d
                pltpu.async_copy(slot((k + 2) % 3), out_hbm.at[row], osem)
                return carry

            lax.fori_loop(1, _KC, body, 0)

            wait_gsem()
            row = (sid + _NS * (_KC - 1)) * _EMB + d
            pltpu.async_copy(slot((_KC - 1) % 3), out_hbm.at[row], osem)

            pltpu.make_async_copy(ghalf, dump_hbm.at[2, pl.ds(0, _HB)],
                                  osem2).wait()
            pltpu.async_copy(shr.at[ihalf], ghalf, gsem)
            pltpu.make_async_copy(tab_hbm.at[0, pl.ds(0, _HB)], ghalf,
                                  gsem).wait()
            pltpu.async_copy(ghalf,
                             out_hbm.at[lh * _EMB + d, pl.ds(bho, _HB)],
                             osem2)

        drain_osem()
        drain_osem()
        pltpu.make_async_copy(ghalf, dump_hbm.at[2, pl.ds(0, _HB)],
                              osem2).wait()

    return gather_kernel


def kernel(input, table):
    idxP = input.T
    tP = table.T
    tailT = jnp.pad(tP[:, _NS * _LCH:], ((0, 0), (0, _VP - _V)))
    out2d, _ = _build()(idxP, tP, tailT)
    out3 = out2d.reshape(_L, _EMB, _B)
    return jnp.transpose(out3, (2, 0, 1))
